# Initial kernel scaffold; baseline (speedup 1.0000x reference)
import jax
import jax.numpy as jnp
from jax.experimental import pallas as pl


def kernel(pressure, mass_flux, quality, mass_flux_grid, quality_grid, pressure_grid, data):
    raise NotImplementedError("write your pallas kernel here")


# speedup vs baseline: 280.9418x; 280.9418x over previous
import functools

import jax
import jax.numpy as jnp
from jax import lax
from jax.experimental import pallas as pl
from jax.experimental.pallas import tpu as pltpu
from jax.experimental.pallas import tpu_sc as plsc

N = 1048576
G0, G1, G2 = 64, 96, 64
NC, NS, L = 2, 16, 16
NW = NC * NS
P = N // NW
C = 1024
NV = C // L
NCHUNK = P // C

_STEPS_64 = (32, 16, 8, 4, 2, 1)
_STEPS_96 = (64, 32, 16, 8, 4, 2, 1)


def _search(grid_ref, n, steps, x):
    c = jnp.zeros((L,), jnp.int32)
    for b in steps:
        cand = c + b
        probe = jnp.minimum(cand - 1, n - 1)
        g = plsc.load_gather(grid_ref, [probe])
        ok = (g <= x) & (cand <= n)
        c = jnp.where(ok, cand, c)
    i = jnp.clip(c - 1, 0, n - 2)
    glo = plsc.load_gather(grid_ref, [i])
    ghi = plsc.load_gather(grid_ref, [i + 1])
    t = (x - glo) / (ghi - glo)
    return i, t


def _make_kernel():
    mesh = plsc.VectorSubcoreMesh(core_axis_name="c", subcore_axis_name="s")

    @functools.partial(
        pl.kernel,
        mesh=mesh,
        compiler_params=pltpu.CompilerParams(needs_layout_passes=False),
        out_type=jax.ShapeDtypeStruct((N,), jnp.float32),
        scratch_types=[
            pltpu.VMEM((G0,), jnp.float32),
            pltpu.VMEM((G1,), jnp.float32),
            pltpu.VMEM((G2,), jnp.float32),
            pltpu.VMEM((C,), jnp.float32),
            pltpu.VMEM((C,), jnp.float32),
            pltpu.VMEM((C,), jnp.float32),
            pltpu.VMEM((8, C), jnp.float32),
            pltpu.VMEM((C,), jnp.float32),
            pltpu.SemaphoreType.DMA,
        ] + [pltpu.VMEM((C,), jnp.int32) for _ in range(8)]
          + [pltpu.VMEM((C,), jnp.float32) for _ in range(8)],
    )
    def lut_kernel(p_hbm, mf_hbm, q_hbm, g0_hbm, g1_hbm, g2_hbm, data_hbm,
                   out_hbm,
                   g0_v, g1_v, g2_v, mf_v, q_v, p_v, w_v,
                   out_v, sem, *corner_refs):
        idx_refs = corner_refs[:8]
        vals_refs = corner_refs[8:]
        wid = lax.axis_index("s") * NC + lax.axis_index("c")
        base = wid * P
        pltpu.sync_copy(g0_hbm, g0_v)
        pltpu.sync_copy(g1_hbm, g1_v)
        pltpu.sync_copy(g2_hbm, g2_v)

        def chunk_body(ci, carry):
            off = base + ci * C
            pltpu.sync_copy(mf_hbm.at[pl.ds(off, C)], mf_v)
            pltpu.sync_copy(q_hbm.at[pl.ds(off, C)], q_v)
            pltpu.sync_copy(p_hbm.at[pl.ds(off, C)], p_v)

            def vec_body(vi, carry2):
                s = vi * L
                mf = mf_v[pl.ds(s, L)]
                qq = q_v[pl.ds(s, L)]
                pp = p_v[pl.ds(s, L)]
                i0, t0 = _search(g0_v, G0, _STEPS_64, mf)
                i1, t1 = _search(g1_v, G1, _STEPS_96, qq)
                i2, t2 = _search(g2_v, G2, _STEPS_64, pp)
                fbase = i0 * (G1 * G2) + i1 * G2 + i2
                u0 = 1.0 - t0
                u1 = 1.0 - t1
                u2 = 1.0 - t2
                for a in (0, 1):
                    w0 = t0 if a else u0
                    for b in (0, 1):
                        w01 = w0 * (t1 if b else u1)
                        for cc in (0, 1):
                            k = a * 4 + b * 2 + cc
                            idx_refs[k][pl.ds(s, L)] = (
                                fbase + (a * (G1 * G2) + b * G2 + cc))
                            w_v[k, pl.ds(s, L)] = w01 * (t2 if cc else u2)
                return carry2

            lax.fori_loop(0, NV, vec_body, 0)

            copies = [
                pltpu.async_copy(data_hbm.at[idx_refs[k]], vals_refs[k], sem)
                for k in range(8)
            ]
            for cp in copies:
                cp.wait()

            def acc_body(vi, carry2):
                s = vi * L
                acc = w_v[0, pl.ds(s, L)] * vals_refs[0][pl.ds(s, L)]
                for k in range(1, 8):
                    acc = acc + w_v[k, pl.ds(s, L)] * vals_refs[k][pl.ds(s, L)]
                out_v[pl.ds(s, L)] = acc
                return carry2

            lax.fori_loop(0, NV, acc_body, 0)
            pltpu.sync_copy(out_v, out_hbm.at[pl.ds(off, C)])
            return carry

        lax.fori_loop(0, NCHUNK, chunk_body, 0)

    return lut_kernel


_LUT_KERNEL = _make_kernel()


@jax.jit
def kernel(pressure, mass_flux, quality, mass_flux_grid, quality_grid,
           pressure_grid, data):
    return _LUT_KERNEL(pressure, mass_flux, quality, mass_flux_grid,
                       quality_grid, pressure_grid, data.reshape(-1))


# speedup vs baseline: 424.0571x; 1.5094x over previous
import functools

import jax
import jax.numpy as jnp
from jax import lax
from jax.experimental import pallas as pl
from jax.experimental.pallas import tpu as pltpu
from jax.experimental.pallas import tpu_sc as plsc

N = 1048576
G0, G1, G2 = 64, 96, 64
NC, NS, L = 2, 16, 16
NW = NC * NS
P = N // NW
C = 1024
NV = C // L
NCHUNK = P // C

_STEPS_64 = (32, 16, 8, 4, 2, 1)
_STEPS_96 = (64, 32, 16, 8, 4, 2, 1)


def _search(grid_ref, n, steps, x):
    c = jnp.zeros((L,), jnp.int32)
    for b in steps:
        cand = c + b
        probe = jnp.minimum(cand - 1, n - 1)
        g = plsc.load_gather(grid_ref, [probe])
        ok = (g <= x) & (cand <= n)
        c = jnp.where(ok, cand, c)
    i = jnp.clip(c - 1, 0, n - 2)
    glo = plsc.load_gather(grid_ref, [i])
    ghi = plsc.load_gather(grid_ref, [i + 1])
    t = (x - glo) / (ghi - glo)
    return i, t


def _make_kernel():
    mesh = plsc.VectorSubcoreMesh(core_axis_name="c", subcore_axis_name="s")

    @functools.partial(
        pl.kernel,
        mesh=mesh,
        compiler_params=pltpu.CompilerParams(needs_layout_passes=False),
        out_type=jax.ShapeDtypeStruct((N,), jnp.float32),
        scratch_types=[
            pltpu.VMEM((G0,), jnp.float32),
            pltpu.VMEM((G1,), jnp.float32),
            pltpu.VMEM((G2,), jnp.float32),
            pltpu.VMEM((C,), jnp.float32),
            pltpu.VMEM((C,), jnp.float32),
            pltpu.VMEM((C,), jnp.float32),
            pltpu.VMEM((8, C), jnp.float32),
            pltpu.VMEM((C,), jnp.float32),
            pltpu.SemaphoreType.DMA,
            pltpu.VMEM_SHARED((G0 * G1 * G2,), jnp.float32),
        ] + [pltpu.VMEM((C,), jnp.int32) for _ in range(8)]
          + [pltpu.VMEM((C,), jnp.float32) for _ in range(8)],
    )
    def lut_kernel(p_hbm, mf_hbm, q_hbm, g0_hbm, g1_hbm, g2_hbm, data_hbm,
                   out_hbm,
                   g0_v, g1_v, g2_v, mf_v, q_v, p_v, w_v,
                   out_v, sem, data_sh, *corner_refs):
        idx_refs = corner_refs[:8]
        vals_refs = corner_refs[8:]
        sid = lax.axis_index("s")
        seg = (G0 * G1 * G2) // NS
        pltpu.sync_copy(data_hbm.at[pl.ds(sid * seg, seg)],
                        data_sh.at[pl.ds(sid * seg, seg)])
        plsc.subcore_barrier()
        wid = lax.axis_index("s") * NC + lax.axis_index("c")
        base = wid * P
        pltpu.sync_copy(g0_hbm, g0_v)
        pltpu.sync_copy(g1_hbm, g1_v)
        pltpu.sync_copy(g2_hbm, g2_v)

        def chunk_body(ci, carry):
            off = base + ci * C
            pltpu.sync_copy(mf_hbm.at[pl.ds(off, C)], mf_v)
            pltpu.sync_copy(q_hbm.at[pl.ds(off, C)], q_v)
            pltpu.sync_copy(p_hbm.at[pl.ds(off, C)], p_v)

            def vec_body(vi, carry2):
                s = vi * L
                mf = mf_v[pl.ds(s, L)]
                qq = q_v[pl.ds(s, L)]
                pp = p_v[pl.ds(s, L)]
                i0, t0 = _search(g0_v, G0, _STEPS_64, mf)
                i1, t1 = _search(g1_v, G1, _STEPS_96, qq)
                i2, t2 = _search(g2_v, G2, _STEPS_64, pp)
                fbase = i0 * (G1 * G2) + i1 * G2 + i2
                u0 = 1.0 - t0
                u1 = 1.0 - t1
                u2 = 1.0 - t2
                for a in (0, 1):
                    w0 = t0 if a else u0
                    for b in (0, 1):
                        w01 = w0 * (t1 if b else u1)
                        for cc in (0, 1):
                            k = a * 4 + b * 2 + cc
                            idx_refs[k][pl.ds(s, L)] = (
                                fbase + (a * (G1 * G2) + b * G2 + cc))
                            w_v[k, pl.ds(s, L)] = w01 * (t2 if cc else u2)
                return carry2

            lax.fori_loop(0, NV, vec_body, 0)

            copies = [
                pltpu.async_copy(data_sh.at[idx_refs[k]], vals_refs[k], sem)
                for k in range(8)
            ]
            for cp in copies:
                cp.wait()

            def acc_body(vi, carry2):
                s = vi * L
                acc = w_v[0, pl.ds(s, L)] * vals_refs[0][pl.ds(s, L)]
                for k in range(1, 8):
                    acc = acc + w_v[k, pl.ds(s, L)] * vals_refs[k][pl.ds(s, L)]
                out_v[pl.ds(s, L)] = acc
                return carry2

            lax.fori_loop(0, NV, acc_body, 0)
            pltpu.sync_copy(out_v, out_hbm.at[pl.ds(off, C)])
            return carry

        lax.fori_loop(0, NCHUNK, chunk_body, 0)

    return lut_kernel


_LUT_KERNEL = _make_kernel()


@jax.jit
def kernel(pressure, mass_flux, quality, mass_flux_grid, quality_grid,
           pressure_grid, data):
    return _LUT_KERNEL(pressure, mass_flux, quality, mass_flux_grid,
                       quality_grid, pressure_grid, data.reshape(-1))


# speedup vs baseline: 592.9552x; 1.3983x over previous
import functools

import jax
import jax.numpy as jnp
from jax import lax
from jax.experimental import pallas as pl
from jax.experimental.pallas import tpu as pltpu
from jax.experimental.pallas import tpu_sc as plsc

N = 1048576
G0, G1, G2 = 64, 96, 64
NC, NS, L = 2, 16, 16
NW = NC * NS
P = N // NW
C = 1024
NV = C // L
NCHUNK = P // C
H = NCHUNK // 2

_STEPS_64 = (32, 16, 8, 4, 2, 1)
_STEPS_96 = (64, 32, 16, 8, 4, 2, 1)


def _search(grid_ref, n, steps, x):
    c = jnp.zeros((L,), jnp.int32)
    for b in steps:
        cand = c + b
        probe = jnp.minimum(cand - 1, n - 1)
        g = plsc.load_gather(grid_ref, [probe])
        ok = (g <= x) & (cand <= n)
        c = jnp.where(ok, cand, c)
    i = jnp.clip(c - 1, 0, n - 2)
    glo = plsc.load_gather(grid_ref, [i])
    ghi = plsc.load_gather(grid_ref, [i + 1])
    t = (x - glo) / (ghi - glo)
    return i, t


def _buf_types():
    return ([pltpu.VMEM((C,), jnp.float32) for _ in range(3)]
            + [pltpu.VMEM((8, C), jnp.float32)]
            + [pltpu.VMEM((C,), jnp.float32)]
            + [pltpu.VMEM((C,), jnp.int32) for _ in range(8)]
            + [pltpu.VMEM((C,), jnp.float32) for _ in range(8)])


def _make_kernel():
    mesh = plsc.VectorSubcoreMesh(core_axis_name="c", subcore_axis_name="s")

    @functools.partial(
        pl.kernel,
        mesh=mesh,
        compiler_params=pltpu.CompilerParams(needs_layout_passes=False),
        out_type=jax.ShapeDtypeStruct((N,), jnp.float32),
        scratch_types=[
            pltpu.VMEM((G0,), jnp.float32),
            pltpu.VMEM((G1,), jnp.float32),
            pltpu.VMEM((G2,), jnp.float32),
            pltpu.VMEM_SHARED((G0 * G1 * G2,), jnp.float32),
        ] + [pltpu.SemaphoreType.DMA for _ in range(6)]
          + _buf_types() + _buf_types(),
    )
    def lut_kernel(p_hbm, mf_hbm, q_hbm, g0_hbm, g1_hbm, g2_hbm, data_hbm,
                   out_hbm, *scr):
        g0_v, g1_v, g2_v, data_sh = scr[0:4]
        sem_in = scr[4:6]
        sem_g = scr[6:8]
        sem_out = scr[8:10]
        per = 21
        bufs = []
        for b in range(2):
            s = scr[10 + b * per: 10 + (b + 1) * per]
            bufs.append(dict(mf=s[0], q=s[1], p=s[2], w=s[3], out=s[4],
                             idx=s[5:13], vals=s[13:21]))

        wid = lax.axis_index("s") * NC + lax.axis_index("c")
        base = wid * P

        pltpu.sync_copy(g0_hbm, g0_v)
        pltpu.sync_copy(g1_hbm, g1_v)
        pltpu.sync_copy(g2_hbm, g2_v)
        sid = lax.axis_index("s")
        seg = (G0 * G1 * G2) // NS
        pltpu.sync_copy(data_hbm.at[pl.ds(sid * seg, seg)],
                        data_sh.at[pl.ds(sid * seg, seg)])
        plsc.subcore_barrier()

        def stage_in(ci, b):
            off = base + ci * C
            B = bufs[b]
            pltpu.async_copy(mf_hbm.at[pl.ds(off, C)], B['mf'], sem_in[b])
            pltpu.async_copy(q_hbm.at[pl.ds(off, C)], B['q'], sem_in[b])
            pltpu.async_copy(p_hbm.at[pl.ds(off, C)], B['p'], sem_in[b])

        def wait_in(b):
            B = bufs[b]
            pltpu.make_async_copy(mf_hbm.at[pl.ds(0, C)], B['mf'], sem_in[b]).wait()
            pltpu.make_async_copy(q_hbm.at[pl.ds(0, C)], B['q'], sem_in[b]).wait()
            pltpu.make_async_copy(p_hbm.at[pl.ds(0, C)], B['p'], sem_in[b]).wait()

        def compute_idx(b):
            B = bufs[b]
            idx_refs, w_v = B['idx'], B['w']
            mf_v, q_v, p_v = B['mf'], B['q'], B['p']

            def vec_body(vi, carry):
                s = vi * L
                mf = mf_v[pl.ds(s, L)]
                qq = q_v[pl.ds(s, L)]
                pp = p_v[pl.ds(s, L)]
                i0, t0 = _search(g0_v, G0, _STEPS_64, mf)
                i1, t1 = _search(g1_v, G1, _STEPS_96, qq)
                i2, t2 = _search(g2_v, G2, _STEPS_64, pp)
                fbase = i0 * (G1 * G2) + i1 * G2 + i2
                u0 = 1.0 - t0
                u1 = 1.0 - t1
                u2 = 1.0 - t2
                for a in (0, 1):
                    w0 = t0 if a else u0
                    for bb in (0, 1):
                        w01 = w0 * (t1 if bb else u1)
                        for cc in (0, 1):
                            k = a * 4 + bb * 2 + cc
                            idx_refs[k][pl.ds(s, L)] = (
                                fbase + (a * (G1 * G2) + bb * G2 + cc))
                            w_v[k, pl.ds(s, L)] = w01 * (t2 if cc else u2)
                return carry

            lax.fori_loop(0, NV, vec_body, 0)

        def fire_gathers(b):
            B = bufs[b]
            for k in range(8):
                pltpu.async_copy(data_sh.at[B['idx'][k]], B['vals'][k],
                                 sem_g[b])

        def wait_gathers(b):
            B = bufs[b]
            for k in range(8):
                pltpu.make_async_copy(data_sh.at[B['idx'][k]], B['vals'][k],
                                      sem_g[b]).wait()

        def drain_out(b):
            B = bufs[b]
            pltpu.make_async_copy(B['out'], out_hbm.at[pl.ds(0, C)],
                                  sem_out[b]).wait()

        def accumulate(b):
            B = bufs[b]
            w_v, vals_refs, out_v = B['w'], B['vals'], B['out']

            def acc_body(vi, carry):
                s = vi * L
                acc = w_v[0, pl.ds(s, L)] * vals_refs[0][pl.ds(s, L)]
                for k in range(1, 8):
                    acc = acc + w_v[k, pl.ds(s, L)] * vals_refs[k][pl.ds(s, L)]
                out_v[pl.ds(s, L)] = acc
                return carry

            lax.fori_loop(0, NV, acc_body, 0)

        def fire_out(ci, b):
            B = bufs[b]
            pltpu.async_copy(B['out'], out_hbm.at[pl.ds(base + ci * C, C)],
                             sem_out[b])

        stage_in(0, 0)
        stage_in(1, 1)
        wait_in(0)
        compute_idx(0)
        fire_gathers(0)

        def body(i, carry):
            c0 = 2 * i

            @pl.when(i + 1 < H)
            def _():
                stage_in(c0 + 2, 0)

            wait_in(1)
            compute_idx(1)
            fire_gathers(1)

            @pl.when(i + 1 < H)
            def _():
                stage_in(c0 + 3, 1)

            wait_gathers(0)

            @pl.when(i >= 1)
            def _():
                drain_out(0)

            accumulate(0)
            fire_out(c0, 0)

            @pl.when(i + 1 < H)
            def _():
                wait_in(0)
                compute_idx(0)
                fire_gathers(0)

            wait_gathers(1)

            @pl.when(i >= 1)
            def _():
                drain_out(1)

            accumulate(1)
            fire_out(c0 + 1, 1)
            return carry

        lax.fori_loop(0, H, body, 0)
        drain_out(0)
        drain_out(1)

    return lut_kernel


_LUT_KERNEL = _make_kernel()


@jax.jit
def kernel(pressure, mass_flux, quality, mass_flux_grid, quality_grid,
           pressure_grid, data):
    return _LUT_KERNEL(pressure, mass_flux, quality, mass_flux_grid,
                       quality_grid, pressure_grid, data.reshape(-1))
